# baseline (speedup 1.0000x reference)
import functools

import jax
import jax.numpy as jnp
from jax import lax
from jax.experimental import pallas as pl
from jax.experimental.pallas import tpu as pltpu
from jax.experimental.pallas import tpu_sc as plsc

VOCAB = 1000000
EMBED_DIM = 64
BATCH = 16384
SEQ = 50

NC = 2
NS = 16
NW = NC * NS

SEQ_PAD = 56
EMBED_PAD = 128
BPW = BATCH // NW
CHUNK = 256
QS = CHUNK // 128
HS = BPW // CHUNK
N_CHUNKS = SEQ * HS
NB = 4


def _gather_kernel(table, idx, out, idx_v, rows_a, rows_b, rows_c, rows_d,
                   gsem_a, gsem_b, gsem_c, gsem_d,
                   wsem_a, wsem_b, wsem_c, wsem_d):
    wid = lax.axis_index("s") * NC + lax.axis_index("c")
    b0 = wid * BPW

    rows = (rows_a, rows_b, rows_c, rows_d)
    gsem = (gsem_a, gsem_b, gsem_c, gsem_d)
    wsem = (wsem_a, wsem_b, wsem_c, wsem_d)

    pltpu.sync_copy(idx.at[pl.ds(0, SEQ), pl.ds(b0, BPW)], idx_v)

    def fire_gathers(g, b):
        s = g // HS
        h = g - s * HS
        for q in range(QS):
            pltpu.async_copy(
                table.at[idx_v.at[s, pl.ds(h * CHUNK + q * 128, 128)]],
                rows[b].at[pl.ds(q * 128, 128)],
                gsem[b],
            )

    def drain_gathers(b):
        pltpu.make_async_copy(table.at[pl.ds(0, CHUNK)], rows[b],
                              gsem[b]).wait()

    def fire_writeback(g, b):
        s = g // HS
        h = g - s * HS
        pltpu.async_copy(
            rows[b],
            out.at[pl.ds(b0 + h * CHUNK, CHUNK), s, pl.ds(0, EMBED_DIM)],
            wsem[b],
        )

    def drain_writeback(b):
        pltpu.make_async_copy(
            rows[b],
            out.at[pl.ds(b0, CHUNK), 0, pl.ds(0, EMBED_DIM)],
            wsem[b],
        ).wait()

    for b in range(NB):
        fire_gathers(b, b)

    @pl.loop(0, N_CHUNKS, step=NB)
    def _step(g0):
        for b in range(NB):
            g = g0 + b
            drain_gathers(b)
            fire_writeback(g, b)
            nxt = g + NB

            @pl.when(nxt < N_CHUNKS)
            def _():
                drain_writeback(b)
                fire_gathers(nxt, b)

    for b in range(NB):
        drain_writeback(b)


@jax.jit
def _embedding_gather(xt, table):
    mesh = plsc.VectorSubcoreMesh(core_axis_name="c", subcore_axis_name="s")
    run = functools.partial(
        pl.kernel,
        mesh=mesh,
        out_type=jax.ShapeDtypeStruct((BATCH, SEQ_PAD, EMBED_PAD), jnp.float32),
        scratch_types=[
            pltpu.VMEM((SEQ, BPW), jnp.int32),
            pltpu.VMEM((CHUNK, EMBED_DIM), jnp.float32),
            pltpu.VMEM((CHUNK, EMBED_DIM), jnp.float32),
            pltpu.VMEM((CHUNK, EMBED_DIM), jnp.float32),
            pltpu.VMEM((CHUNK, EMBED_DIM), jnp.float32),
            pltpu.SemaphoreType.DMA,
            pltpu.SemaphoreType.DMA,
            pltpu.SemaphoreType.DMA,
            pltpu.SemaphoreType.DMA,
            pltpu.SemaphoreType.DMA,
            pltpu.SemaphoreType.DMA,
            pltpu.SemaphoreType.DMA,
            pltpu.SemaphoreType.DMA,
        ],
        compiler_params=pltpu.CompilerParams(use_tc_tiling_on_sc=False),
    )(_gather_kernel)
    return run(table, xt)


def kernel(x, embedding):
    xt = x.T.astype(jnp.int32)
    out = _embedding_gather(xt, embedding)
    return out[:, :SEQ, :EMBED_DIM]
